# baseline (speedup 1.0000x reference)
import functools

import jax
import jax.numpy as jnp
import numpy as np
from jax import lax
from jax.experimental import pallas as pl
from jax.experimental.pallas import tpu as pltpu
from jax.experimental.pallas import tpu_sc as plsc

S, D = 2048, 768
HQ, HKV, HD = 12, 4, 64
E, DFF = 8, 3072
BS = 256
NS = S // BS
NF = 4
FB = DFF // NF
NP = 2 * S
RG = 256
NT = NP // RG
NI = NT + E
NEG = -1e30


def _rope_tables(n_cols):
    half = n_cols // 2
    inv = 1.0 / (10000.0 ** (np.arange(0, HD, 2, dtype=np.float64) / HD))
    t = np.arange(S, dtype=np.float64)[:, None]
    f = np.tile(t * inv[None, :], (1, half // 32))
    f = np.concatenate([f, f], axis=1)
    return (jnp.asarray(np.cos(f), jnp.float32),
            jnp.asarray(np.sin(f), jnp.float32))


def _deint_perm(n_heads):
    cols = []
    for halfsel in (0, 1):
        for h in range(n_heads):
            for i in range(32):
                cols.append(h * HD + 2 * i + halfsel)
    return np.asarray(cols, np.int32)


def _qkv_body(x_ref, n1_ref, qw_ref, kw_ref, vw_ref, cq_ref, sq_ref,
              ck_ref, sk_ref, q_ref, k_ref, v_ref):
    x = x_ref[...]
    h = x * lax.rsqrt(jnp.mean(x * x, axis=-1, keepdims=True) + 1e-8)
    h = h * n1_ref[...]
    hp = lax.Precision.HIGHEST
    q = jnp.dot(h, qw_ref[...], preferred_element_type=jnp.float32,
                precision=hp)
    k = jnp.dot(h, kw_ref[...], preferred_element_type=jnp.float32,
                precision=hp)
    v = jnp.dot(h, vw_ref[...], preferred_element_type=jnp.float32,
                precision=hp)
    qe, qo = q[:, :D // 2], q[:, D // 2:]
    qsw = jnp.concatenate([-qo, qe], axis=1)
    q_ref[...] = q * cq_ref[...] + qsw * sq_ref[...]
    kd = HKV * HD
    ke, ko = k[:, :kd // 2], k[:, kd // 2:]
    ksw = jnp.concatenate([-ko, ke], axis=1)
    k_ref[...] = k * ck_ref[...] + ksw * sk_ref[...]
    v_ref[...] = v


def _qkv_call(x2d, n1, qwp, kwp, vw, cq, sq, ck, sk):
    kd = HKV * HD
    return pl.pallas_call(
        _qkv_body,
        grid=(NS,),
        in_specs=[
            pl.BlockSpec((BS, D), lambda i: (i, 0)),
            pl.BlockSpec((1, D), lambda i: (0, 0)),
            pl.BlockSpec((D, D), lambda i: (0, 0)),
            pl.BlockSpec((D, kd), lambda i: (0, 0)),
            pl.BlockSpec((D, kd), lambda i: (0, 0)),
            pl.BlockSpec((BS, D), lambda i: (i, 0)),
            pl.BlockSpec((BS, D), lambda i: (i, 0)),
            pl.BlockSpec((BS, kd), lambda i: (i, 0)),
            pl.BlockSpec((BS, kd), lambda i: (i, 0)),
        ],
        out_specs=[
            pl.BlockSpec((BS, D), lambda i: (i, 0)),
            pl.BlockSpec((BS, kd), lambda i: (i, 0)),
            pl.BlockSpec((BS, kd), lambda i: (i, 0)),
        ],
        out_shape=[
            jax.ShapeDtypeStruct((S, D), jnp.float32),
            jax.ShapeDtypeStruct((S, kd), jnp.float32),
            jax.ShapeDtypeStruct((S, kd), jnp.float32),
        ],
    )(x2d, n1.reshape(1, D), qwp, kwp, vw, cq, sq, ck, sk)


def _attn_body(q_ref, k_ref, v_ref, o_ref):
    qb = pl.program_id(1)
    q = q_ref[0]
    k = k_ref[0]
    v = v_ref[0]
    s = lax.dot_general(q, k, (((1,), (1,)), ((), ())),
                        preferred_element_type=jnp.float32,
                        precision=lax.Precision.HIGHEST)
    s = s * (1.0 / np.sqrt(HD))
    qi = qb * BS + lax.broadcasted_iota(jnp.int32, (BS, S), 0)
    ki = lax.broadcasted_iota(jnp.int32, (BS, S), 1)
    s = jnp.where(ki <= qi, s, NEG)
    m = jnp.max(s, axis=1, keepdims=True)
    p = jnp.exp(s - m)
    l = jnp.sum(p, axis=1, keepdims=True)
    o = jnp.dot(p, v, preferred_element_type=jnp.float32,
                precision=lax.Precision.HIGHEST)
    o_ref[0] = o / l


def _attn_call(qh, kh, vh):
    rep = HQ // HKV
    return pl.pallas_call(
        _attn_body,
        grid=(HQ, NS),
        in_specs=[
            pl.BlockSpec((1, BS, HD), lambda h, i: (h, i, 0)),
            pl.BlockSpec((1, S, HD), lambda h, i: (h // rep, 0, 0)),
            pl.BlockSpec((1, S, HD), lambda h, i: (h // rep, 0, 0)),
        ],
        out_specs=pl.BlockSpec((1, BS, HD), lambda h, i: (h, i, 0)),
        out_shape=jax.ShapeDtypeStruct((HQ, S, HD), jnp.float32),
    )(qh, kh, vh)


def _oproj_body(x_ref, a_ref, ow_ref, o_ref):
    o_ref[...] = x_ref[...] + jnp.dot(a_ref[...], ow_ref[...],
                                      preferred_element_type=jnp.float32,
                                      precision=lax.Precision.HIGHEST)


def _oproj_call(x2d, attn2d, ow):
    return pl.pallas_call(
        _oproj_body,
        grid=(NS,),
        in_specs=[
            pl.BlockSpec((BS, D), lambda i: (i, 0)),
            pl.BlockSpec((BS, D), lambda i: (i, 0)),
            pl.BlockSpec((D, D), lambda i: (0, 0)),
        ],
        out_specs=pl.BlockSpec((BS, D), lambda i: (i, 0)),
        out_shape=jax.ShapeDtypeStruct((S, D), jnp.float32),
    )(x2d, attn2d, ow)


def _router_body(x_ref, n2_ref, gw_ref, gb_ref, nw_ref, nb_ref, noise_ref,
                 h2_ref, w_ref, aux_ref, pos_ref, ie_ref, it_ref,
                 ilo_ref, ihi_ref):
    x = x_ref[...]
    h2 = x * lax.rsqrt(jnp.mean(x * x, axis=-1, keepdims=True) + 1e-8)
    h2 = h2 * n2_ref[...]
    h2_ref[...] = h2
    hp = lax.Precision.HIGHEST
    logits = (jnp.dot(h2, gw_ref[...], preferred_element_type=jnp.float32,
                      precision=hp)
              + gb_ref[...] + noise_ref[...]
              + jnp.dot(h2, nw_ref[...], preferred_element_type=jnp.float32,
                        precision=hp)
              + nb_ref[...])
    ei = lax.broadcasted_iota(jnp.int32, (S, E), 1)
    m1 = jnp.max(logits, axis=1, keepdims=True)
    i1 = jnp.min(jnp.where(logits == m1, ei, E), axis=1, keepdims=True)
    oh1 = (ei == i1)
    lm = jnp.where(oh1, NEG, logits)
    m2 = jnp.max(lm, axis=1, keepdims=True)
    i2 = jnp.min(jnp.where(lm == m2, ei, E), axis=1, keepdims=True)
    oh2 = (ei == i2)
    sel = oh1 | oh2
    z = jnp.where(sel, jnp.exp(logits - m1), 0.0)
    scores = z / jnp.sum(z, axis=1, keepdims=True)
    w1 = jnp.sum(jnp.where(oh1, scores, 0.0), axis=1, keepdims=True)
    w2 = jnp.sum(jnp.where(oh2, scores, 0.0), axis=1, keepdims=True)
    w_ref[...] = jnp.where(ei == 0, w1, 0.0) + jnp.where(ei == 1, w2, 0.0)
    imp = jnp.mean(scores, axis=0, keepdims=True)
    u = 1.0 / E
    aux_ref[...] = jnp.full(
        (1, 1), jnp.sum(u * (jnp.log(u) - jnp.log(imp + 1e-8))), jnp.float32)

    o1f = oh1.astype(jnp.float32)
    o2f = oh2.astype(jnp.float32)
    counts = (jnp.sum(o1f, axis=0, keepdims=True)
              + jnp.sum(o2f, axis=0, keepdims=True))
    e0 = lax.broadcasted_iota(jnp.int32, (E, E), 0)
    e1 = lax.broadcasted_iota(jnp.int32, (E, E), 1)
    u8s = (e0 < e1).astype(jnp.float32)
    u8i = (e0 <= e1).astype(jnp.float32)
    off = jnp.dot(counts, u8s, preferred_element_type=jnp.float32, precision=hp)
    gin = jnp.dot(counts, u8i, preferred_element_type=jnp.float32, precision=hp)
    c0 = lax.broadcasted_iota(jnp.int32, (128, 128), 0)
    c1 = lax.broadcasted_iota(jnp.int32, (128, 128), 1)
    ltri = (c1 < c0).astype(jnp.float32)
    carry = jnp.zeros((1, E), jnp.float32)
    nch = S // 128
    for half, ohf in ((0, o1f), (1, o2f)):
        for c in range(nch):
            ch = ohf[c * 128:(c + 1) * 128, :]
            excl = jnp.dot(ltri, ch, preferred_element_type=jnp.float32, precision=hp)
            excl = excl + carry
            posc = jnp.sum((excl + off) * ch, axis=1, keepdims=True)
            pos_ref[half * S + c * 128:half * S + (c + 1) * 128, :] = (
                posc.astype(jnp.int32))
            carry = carry + jnp.sum(ch, axis=0, keepdims=True)

    cnt = counts.astype(jnp.int32)
    offi = off.astype(jnp.int32)
    gini = gin.astype(jnp.int32)
    a = offi // RG
    b = (gini - 1) // RG
    n = jnp.where(cnt > 0, b - a + 1, 0)
    nf = n.astype(jnp.float32)
    sexf = jnp.dot(nf, u8s, preferred_element_type=jnp.float32, precision=hp)
    sex = sexf.astype(jnp.int32)
    sin = sex + n
    total = jnp.sum(n)
    erow = lax.broadcasted_iota(jnp.int32, (1, E), 1)
    last_e = jnp.max(jnp.where(n > 0, erow, -1))
    kk = lax.broadcasted_iota(jnp.int32, (NI, 1), 0)
    eraw = jnp.sum((kk >= sin).astype(jnp.int32), axis=1, keepdims=True)
    ek = jnp.minimum(eraw, last_e)
    ohk = (ek == lax.broadcasted_iota(jnp.int32, (NI, E), 1)).astype(
        jnp.int32)
    aK = jnp.sum(ohk * a, axis=1, keepdims=True)
    sK = jnp.sum(ohk * sex, axis=1, keepdims=True)
    oK = jnp.sum(ohk * offi, axis=1, keepdims=True)
    gK = jnp.sum(ohk * gini, axis=1, keepdims=True)
    b_last = jnp.sum(jnp.where(erow == last_e, b, 0))
    valid = kk < total
    tK = jnp.where(valid, aK + kk - sK, b_last)
    lo = jnp.clip(oK - tK * RG, 0, RG)
    hi = jnp.clip(gK - tK * RG, 0, RG)
    ie_ref[...] = ek
    it_ref[...] = tK
    ilo_ref[...] = jnp.where(valid, lo, 0)
    ihi_ref[...] = jnp.where(valid, hi, 0)


def _router_call(x2d, n2, gw, gb, nw, nb, noise):
    return pl.pallas_call(
        _router_body,
        out_shape=[
            jax.ShapeDtypeStruct((S, D), jnp.float32),
            jax.ShapeDtypeStruct((S, E), jnp.float32),
            jax.ShapeDtypeStruct((1, 1), jnp.float32),
            jax.ShapeDtypeStruct((NP, 1), jnp.int32),
            jax.ShapeDtypeStruct((NI, 1), jnp.int32),
            jax.ShapeDtypeStruct((NI, 1), jnp.int32),
            jax.ShapeDtypeStruct((NI, 1), jnp.int32),
            jax.ShapeDtypeStruct((NI, 1), jnp.int32),
        ],
    )(x2d, n2.reshape(1, D), gw, gb.reshape(1, E), nw, nb.reshape(1, E),
      noise.reshape(S, E))


def _sc_mesh():
    return plsc.VectorSubcoreMesh(core_axis_name="c", subcore_axis_name="s",
                                  num_cores=2, num_subcores=16)


def _sc_scatter_rows(src, pos):
    ch = NP // 32

    @functools.partial(
        pl.kernel,
        out_type=jax.ShapeDtypeStruct((NP, D), jnp.float32),
        mesh=_sc_mesh(),
        scratch_types=[pltpu.VMEM((ch,), jnp.int32),
                       pltpu.VMEM((ch, D), jnp.float32),
                       pltpu.SemaphoreType.DMA],
    )
    def k(src_hbm, pos_hbm, out_hbm, idx_v, rows_v, sem):
        wid = lax.axis_index("s") * 2 + lax.axis_index("c")
        base = wid * ch
        pltpu.sync_copy(pos_hbm.at[pl.ds(base, ch)], idx_v)
        pltpu.sync_copy(src_hbm.at[pl.ds((wid % (S // ch)) * ch, ch)],
                        rows_v)
        pltpu.async_copy(rows_v, out_hbm.at[idx_v], sem).wait()

    return k(src, pos)


def _sc_gather_rows(table, idx):
    nrows = idx.shape[0]
    ch = nrows // 32

    @functools.partial(
        pl.kernel,
        out_type=jax.ShapeDtypeStruct((nrows, D), jnp.float32),
        mesh=_sc_mesh(),
        scratch_types=[pltpu.VMEM((ch,), jnp.int32),
                       pltpu.VMEM((ch, D), jnp.float32),
                       pltpu.SemaphoreType.DMA],
    )
    def k(tab_hbm, idx_hbm, out_hbm, idx_v, rows_v, sem):
        wid = lax.axis_index("s") * 2 + lax.axis_index("c")
        base = wid * ch
        pltpu.sync_copy(idx_hbm.at[pl.ds(base, ch)], idx_v)
        pltpu.async_copy(tab_hbm.at[idx_v], rows_v, sem).wait()
        pltpu.sync_copy(rows_v, out_hbm.at[pl.ds(base, ch)])

    return k(table, idx)


def _gg_body(ie_ref, it_ref, ilo_ref, ihi_ref, xs_ref, ew1_ref, eb1_ref,
             ew2_ref, eb2_ref, out_ref, acc_ref):
    f = pl.program_id(0)
    k = pl.program_id(1)
    lo = ilo_ref[k]
    hi = ihi_ref[k]
    rows = lax.broadcasted_iota(jnp.int32, (RG, 1), 0)
    valid = (rows >= lo) & (rows < hi)
    pre = jnp.dot(xs_ref[...], ew1_ref[0],
                  preferred_element_type=jnp.float32) + eb1_ref[0]
    hid = pre * jax.nn.sigmoid(pre)
    part = jnp.dot(hid, ew2_ref[0], preferred_element_type=jnp.float32)
    part = part + jnp.where(f == 0, eb2_ref[0], 0.0)
    sl = pl.ds(it_ref[k] * RG, RG)
    prev = jnp.where(f == 0, 0.0, acc_ref[sl, :])
    acc_ref[sl, :] = jnp.where(valid, prev + part, acc_ref[sl, :])
    out_ref[...] = acc_ref[sl, :]


def _gg_call(xs, ew1, eb1, ew2, eb2, ie, it, ilo, ihi):
    grid_spec = pltpu.PrefetchScalarGridSpec(
        num_scalar_prefetch=4,
        grid=(NF, NI),
        in_specs=[
            pl.BlockSpec((RG, D), lambda f, k, ie, it, lo, hi: (it[k], 0)),
            pl.BlockSpec((1, D, FB), lambda f, k, ie, it, lo, hi:
                         (ie[k], 0, f)),
            pl.BlockSpec((1, 1, FB), lambda f, k, ie, it, lo, hi:
                         (ie[k], 0, f)),
            pl.BlockSpec((1, FB, D), lambda f, k, ie, it, lo, hi:
                         (ie[k], f, 0)),
            pl.BlockSpec((1, 1, D), lambda f, k, ie, it, lo, hi:
                         (ie[k], 0, 0)),
        ],
        out_specs=pl.BlockSpec((RG, D), lambda f, k, ie, it, lo, hi:
                               (it[k], 0)),
        scratch_shapes=[pltpu.VMEM((NP, D), jnp.float32)],
    )
    return pl.pallas_call(
        _gg_body,
        grid_spec=grid_spec,
        out_shape=jax.ShapeDtypeStruct((NP, D), jnp.float32),
    )(ie, it, ilo, ihi, xs, ew1, eb1.reshape(E, 1, DFF), ew2,
      eb2.reshape(E, 1, D))


def _comb_body(x_ref, o1_ref, o2_ref, w_ref, out_ref):
    ei = lax.broadcasted_iota(jnp.int32, (BS, E), 1)
    wp = w_ref[...]
    w1 = jnp.sum(jnp.where(ei == 0, wp, 0.0), axis=1, keepdims=True)
    w2 = jnp.sum(jnp.where(ei == 1, wp, 0.0), axis=1, keepdims=True)
    out_ref[...] = x_ref[...] + w1 * o1_ref[...] + w2 * o2_ref[...]


def _comb_call(x2d, og, wpad):
    return pl.pallas_call(
        _comb_body,
        grid=(NS,),
        in_specs=[
            pl.BlockSpec((BS, D), lambda i: (i, 0)),
            pl.BlockSpec((BS, D), lambda i: (i, 0)),
            pl.BlockSpec((BS, D), lambda i: (i + NS, 0)),
            pl.BlockSpec((BS, E), lambda i: (i, 0)),
        ],
        out_specs=pl.BlockSpec((BS, D), lambda i: (i, 0)),
        out_shape=jax.ShapeDtypeStruct((S, D), jnp.float32),
    )(x2d, og, og, wpad)


_QPERM = _deint_perm(HQ)
_KPERM = _deint_perm(HKV)


def kernel(x, norm1_w, q_w, k_w, v_w, o_w, norm2_w, gate_w, gate_b,
           nz_w, nz_b, ew1, eb1, ew2, eb2, noise):
    x2d = x.reshape(S, D)
    cq, sq = _rope_tables(D)
    ck, sk = _rope_tables(HKV * HD)
    qwp = q_w[:, _QPERM]
    kwp = k_w[:, _KPERM]

    q, k, v = _qkv_call(x2d, norm1_w, qwp, kwp, v_w, cq, sq, ck, sk)
    qh = q.reshape(S, 2, HQ, 32).transpose(2, 0, 1, 3).reshape(HQ, S, HD)
    kh = k.reshape(S, 2, HKV, 32).transpose(2, 0, 1, 3).reshape(HKV, S, HD)
    vh = v.reshape(S, HKV, HD).transpose(1, 0, 2)
    attn = _attn_call(qh, kh, vh)
    attn2d = attn.transpose(1, 0, 2).reshape(S, D)
    xa = _oproj_call(x2d, attn2d, o_w)

    h2, wpad, aux, pos, ie, it, ilo, ihi = _router_call(
        xa, norm2_w, gate_w, gate_b, nz_w, nz_b, noise)
    pos1 = pos.reshape(NP)
    xs = _sc_scatter_rows(h2, pos1)
    out_sorted = _gg_call(xs, ew1, eb1, ew2, eb2,
                          ie.reshape(NI), it.reshape(NI),
                          ilo.reshape(NI), ihi.reshape(NI))
    og = _sc_gather_rows(out_sorted, pos1)
    out = _comb_call(xa, og, wpad)
    return out.reshape(1, S, D), aux[0, 0]


# speedup vs baseline: 2.1780x; 2.1780x over previous
import functools

import jax
import jax.numpy as jnp
import numpy as np
from jax import lax
from jax.experimental import pallas as pl
from jax.experimental.pallas import tpu as pltpu
from jax.experimental.pallas import tpu_sc as plsc

S, D = 2048, 768
HQ, HKV, HD = 12, 4, 64
E, DFF = 8, 3072
BS = 256
NS = S // BS
NF = 4
FB = DFF // NF
NP = 2 * S
RG = 256
NT = NP // RG
NI = NT + E
NEG = -1e30


def _rope_tables(n_cols):
    half = n_cols // 2
    inv = 1.0 / (10000.0 ** (np.arange(0, HD, 2, dtype=np.float64) / HD))
    t = np.arange(S, dtype=np.float64)[:, None]
    f = np.tile(t * inv[None, :], (1, half // 32))
    f = np.concatenate([f, f], axis=1)
    return (jnp.asarray(np.cos(f), jnp.float32),
            jnp.asarray(np.sin(f), jnp.float32))


def _deint_perm(n_heads):
    cols = []
    for halfsel in (0, 1):
        for h in range(n_heads):
            for i in range(32):
                cols.append(h * HD + 2 * i + halfsel)
    return np.asarray(cols, np.int32)


def _qkv_body(x_ref, n1_ref, qw_ref, kw_ref, vw_ref, cq_ref, sq_ref,
              ck_ref, sk_ref, q_ref, k_ref, v_ref):
    x = x_ref[...]
    h = x * lax.rsqrt(jnp.mean(x * x, axis=-1, keepdims=True) + 1e-8)
    h = h * n1_ref[...]
    q = jnp.dot(h, qw_ref[...], preferred_element_type=jnp.float32)
    k = jnp.dot(h, kw_ref[...], preferred_element_type=jnp.float32)
    v = jnp.dot(h, vw_ref[...], preferred_element_type=jnp.float32)
    qe, qo = q[:, :D // 2], q[:, D // 2:]
    qsw = jnp.concatenate([-qo, qe], axis=1)
    q_ref[...] = q * cq_ref[...] + qsw * sq_ref[...]
    kd = HKV * HD
    ke, ko = k[:, :kd // 2], k[:, kd // 2:]
    ksw = jnp.concatenate([-ko, ke], axis=1)
    k_ref[...] = k * ck_ref[...] + ksw * sk_ref[...]
    v_ref[...] = v


def _qkv_call(x2d, n1, qwp, kwp, vw, cq, sq, ck, sk):
    kd = HKV * HD
    return pl.pallas_call(
        _qkv_body,
        grid=(NS,),
        in_specs=[
            pl.BlockSpec((BS, D), lambda i: (i, 0)),
            pl.BlockSpec((1, D), lambda i: (0, 0)),
            pl.BlockSpec((D, D), lambda i: (0, 0)),
            pl.BlockSpec((D, kd), lambda i: (0, 0)),
            pl.BlockSpec((D, kd), lambda i: (0, 0)),
            pl.BlockSpec((BS, D), lambda i: (i, 0)),
            pl.BlockSpec((BS, D), lambda i: (i, 0)),
            pl.BlockSpec((BS, kd), lambda i: (i, 0)),
            pl.BlockSpec((BS, kd), lambda i: (i, 0)),
        ],
        out_specs=[
            pl.BlockSpec((BS, D), lambda i: (i, 0)),
            pl.BlockSpec((BS, kd), lambda i: (i, 0)),
            pl.BlockSpec((BS, kd), lambda i: (i, 0)),
        ],
        out_shape=[
            jax.ShapeDtypeStruct((S, D), jnp.float32),
            jax.ShapeDtypeStruct((S, kd), jnp.float32),
            jax.ShapeDtypeStruct((S, kd), jnp.float32),
        ],
    )(x2d, n1.reshape(1, D), qwp, kwp, vw, cq, sq, ck, sk)


def _attn_body(q_ref, k_ref, v_ref, o_ref):
    qb = pl.program_id(1)
    q = q_ref[0]
    k = k_ref[0]
    v = v_ref[0]
    s = lax.dot_general(q, k, (((1,), (1,)), ((), ())),
                        preferred_element_type=jnp.float32)
    s = s * (1.0 / np.sqrt(HD))
    qi = qb * BS + lax.broadcasted_iota(jnp.int32, (BS, S), 0)
    ki = lax.broadcasted_iota(jnp.int32, (BS, S), 1)
    s = jnp.where(ki <= qi, s, NEG)
    m = jnp.max(s, axis=1, keepdims=True)
    p = jnp.exp(s - m)
    l = jnp.sum(p, axis=1, keepdims=True)
    o = jnp.dot(p, v, preferred_element_type=jnp.float32)
    o_ref[0] = o / l


def _attn_call(qh, kh, vh):
    rep = HQ // HKV
    return pl.pallas_call(
        _attn_body,
        grid=(HQ, NS),
        in_specs=[
            pl.BlockSpec((1, BS, HD), lambda h, i: (h, i, 0)),
            pl.BlockSpec((1, S, HD), lambda h, i: (h // rep, 0, 0)),
            pl.BlockSpec((1, S, HD), lambda h, i: (h // rep, 0, 0)),
        ],
        out_specs=pl.BlockSpec((1, BS, HD), lambda h, i: (h, i, 0)),
        out_shape=jax.ShapeDtypeStruct((HQ, S, HD), jnp.float32),
    )(qh, kh, vh)


def _oproj_body(x_ref, a_ref, ow_ref, o_ref):
    o_ref[...] = x_ref[...] + jnp.dot(a_ref[...], ow_ref[...],
                                      preferred_element_type=jnp.float32)


def _oproj_call(x2d, attn2d, ow):
    return pl.pallas_call(
        _oproj_body,
        grid=(NS,),
        in_specs=[
            pl.BlockSpec((BS, D), lambda i: (i, 0)),
            pl.BlockSpec((BS, D), lambda i: (i, 0)),
            pl.BlockSpec((D, D), lambda i: (0, 0)),
        ],
        out_specs=pl.BlockSpec((BS, D), lambda i: (i, 0)),
        out_shape=jax.ShapeDtypeStruct((S, D), jnp.float32),
    )(x2d, attn2d, ow)


def _router_body(x_ref, n2_ref, gw_ref, gb_ref, nw_ref, nb_ref, noise_ref,
                 h2_ref, w_ref, aux_ref, pos_ref, ie_ref, it_ref,
                 ilo_ref, ihi_ref):
    x = x_ref[...]
    h2 = x * lax.rsqrt(jnp.mean(x * x, axis=-1, keepdims=True) + 1e-8)
    h2 = h2 * n2_ref[...]
    h2_ref[...] = h2
    hp = lax.Precision.HIGHEST
    h2b = h2.astype(jnp.bfloat16)
    logits = (jnp.dot(h2b, gw_ref[...].astype(jnp.bfloat16),
                      preferred_element_type=jnp.float32)
              + gb_ref[...] + noise_ref[...]
              + jnp.dot(h2b, nw_ref[...].astype(jnp.bfloat16),
                        preferred_element_type=jnp.float32)
              + nb_ref[...])
    ei = lax.broadcasted_iota(jnp.int32, (S, E), 1)
    m1 = jnp.max(logits, axis=1, keepdims=True)
    i1 = jnp.min(jnp.where(logits == m1, ei, E), axis=1, keepdims=True)
    oh1 = (ei == i1)
    lm = jnp.where(oh1, NEG, logits)
    m2 = jnp.max(lm, axis=1, keepdims=True)
    i2 = jnp.min(jnp.where(lm == m2, ei, E), axis=1, keepdims=True)
    oh2 = (ei == i2)
    sel = oh1 | oh2
    z = jnp.where(sel, jnp.exp(logits - m1), 0.0)
    scores = z / jnp.sum(z, axis=1, keepdims=True)
    w1 = jnp.sum(jnp.where(oh1, scores, 0.0), axis=1, keepdims=True)
    w2 = jnp.sum(jnp.where(oh2, scores, 0.0), axis=1, keepdims=True)
    w_ref[...] = jnp.where(ei == 0, w1, 0.0) + jnp.where(ei == 1, w2, 0.0)
    imp = jnp.mean(scores, axis=0, keepdims=True)
    u = 1.0 / E
    aux_ref[...] = jnp.full(
        (1, 1), jnp.sum(u * (jnp.log(u) - jnp.log(imp + 1e-8))), jnp.float32)

    o1f = oh1.astype(jnp.float32)
    o2f = oh2.astype(jnp.float32)
    counts = (jnp.sum(o1f, axis=0, keepdims=True)
              + jnp.sum(o2f, axis=0, keepdims=True))
    e0 = lax.broadcasted_iota(jnp.int32, (E, E), 0)
    e1 = lax.broadcasted_iota(jnp.int32, (E, E), 1)
    u8s = (e0 < e1).astype(jnp.float32)
    u8i = (e0 <= e1).astype(jnp.float32)
    off = jnp.dot(counts, u8s, preferred_element_type=jnp.float32, precision=hp)
    gin = jnp.dot(counts, u8i, preferred_element_type=jnp.float32, precision=hp)
    c0 = lax.broadcasted_iota(jnp.int32, (128, 128), 0)
    c1 = lax.broadcasted_iota(jnp.int32, (128, 128), 1)
    ltri = (c1 < c0).astype(jnp.float32)
    carry = jnp.zeros((1, E), jnp.float32)
    nch = S // 128
    for half, ohf in ((0, o1f), (1, o2f)):
        for c in range(nch):
            ch = ohf[c * 128:(c + 1) * 128, :]
            excl = jnp.dot(ltri, ch, preferred_element_type=jnp.float32, precision=hp)
            excl = excl + carry
            posc = jnp.sum((excl + off) * ch, axis=1, keepdims=True)
            pos_ref[half * S + c * 128:half * S + (c + 1) * 128, :] = (
                posc.astype(jnp.int32))
            carry = carry + jnp.sum(ch, axis=0, keepdims=True)

    cnt = counts.astype(jnp.int32)
    offi = off.astype(jnp.int32)
    gini = gin.astype(jnp.int32)
    a = offi // RG
    b = (gini - 1) // RG
    n = jnp.where(cnt > 0, b - a + 1, 0)
    nf = n.astype(jnp.float32)
    sexf = jnp.dot(nf, u8s, preferred_element_type=jnp.float32, precision=hp)
    sex = sexf.astype(jnp.int32)
    sin = sex + n
    total = jnp.sum(n)
    erow = lax.broadcasted_iota(jnp.int32, (1, E), 1)
    last_e = jnp.max(jnp.where(n > 0, erow, -1))
    kk = lax.broadcasted_iota(jnp.int32, (NI, 1), 0)
    eraw = jnp.sum((kk >= sin).astype(jnp.int32), axis=1, keepdims=True)
    ek = jnp.minimum(eraw, last_e)
    ohk = (ek == lax.broadcasted_iota(jnp.int32, (NI, E), 1)).astype(
        jnp.int32)
    aK = jnp.sum(ohk * a, axis=1, keepdims=True)
    sK = jnp.sum(ohk * sex, axis=1, keepdims=True)
    oK = jnp.sum(ohk * offi, axis=1, keepdims=True)
    gK = jnp.sum(ohk * gini, axis=1, keepdims=True)
    b_last = jnp.sum(jnp.where(erow == last_e, b, 0))
    valid = kk < total
    tK = jnp.where(valid, aK + kk - sK, b_last)
    lo = jnp.clip(oK - tK * RG, 0, RG)
    hi = jnp.clip(gK - tK * RG, 0, RG)
    ie_ref[...] = ek
    it_ref[...] = tK
    ilo_ref[...] = jnp.where(valid, lo, 0)
    ihi_ref[...] = jnp.where(valid, hi, 0)


def _router_call(x2d, n2, gw, gb, nw, nb, noise):
    return pl.pallas_call(
        _router_body,
        out_shape=[
            jax.ShapeDtypeStruct((S, D), jnp.float32),
            jax.ShapeDtypeStruct((S, E), jnp.float32),
            jax.ShapeDtypeStruct((1, 1), jnp.float32),
            jax.ShapeDtypeStruct((NP, 1), jnp.int32),
            jax.ShapeDtypeStruct((NI, 1), jnp.int32),
            jax.ShapeDtypeStruct((NI, 1), jnp.int32),
            jax.ShapeDtypeStruct((NI, 1), jnp.int32),
            jax.ShapeDtypeStruct((NI, 1), jnp.int32),
        ],
    )(x2d, n2.reshape(1, D), gw, gb.reshape(1, E), nw, nb.reshape(1, E),
      noise.reshape(S, E))


def _sc_mesh():
    return plsc.VectorSubcoreMesh(core_axis_name="c", subcore_axis_name="s",
                                  num_cores=2, num_subcores=16)


def _sc_scatter_rows(src, pos):
    ch = NP // 32

    @functools.partial(
        pl.kernel,
        out_type=jax.ShapeDtypeStruct((NP, D), jnp.float32),
        mesh=_sc_mesh(),
        scratch_types=[pltpu.VMEM((ch,), jnp.int32),
                       pltpu.VMEM((ch, D), jnp.float32),
                       pltpu.SemaphoreType.DMA],
    )
    def k(src_hbm, pos_hbm, out_hbm, idx_v, rows_v, sem):
        wid = lax.axis_index("s") * 2 + lax.axis_index("c")
        base = wid * ch
        pltpu.sync_copy(pos_hbm.at[pl.ds(base, ch)], idx_v)
        pltpu.sync_copy(src_hbm.at[pl.ds((wid % (S // ch)) * ch, ch)],
                        rows_v)
        pltpu.async_copy(rows_v, out_hbm.at[idx_v], sem).wait()

    return k(src, pos)


def _sc_gather_rows(table, idx):
    nrows = idx.shape[0]
    ch = nrows // 32

    @functools.partial(
        pl.kernel,
        out_type=jax.ShapeDtypeStruct((nrows, D), jnp.float32),
        mesh=_sc_mesh(),
        scratch_types=[pltpu.VMEM((ch,), jnp.int32),
                       pltpu.VMEM((ch, D), jnp.float32),
                       pltpu.SemaphoreType.DMA],
    )
    def k(tab_hbm, idx_hbm, out_hbm, idx_v, rows_v, sem):
        wid = lax.axis_index("s") * 2 + lax.axis_index("c")
        base = wid * ch
        pltpu.sync_copy(idx_hbm.at[pl.ds(base, ch)], idx_v)
        pltpu.async_copy(tab_hbm.at[idx_v], rows_v, sem).wait()
        pltpu.sync_copy(rows_v, out_hbm.at[pl.ds(base, ch)])

    return k(table, idx)


def _gg_body(ie_ref, it_ref, ilo_ref, ihi_ref, xs_ref, ew1_ref, eb1_ref,
             ew2_ref, eb2_ref, out_ref, acc_ref):
    f = pl.program_id(0)
    k = pl.program_id(1)
    lo = ilo_ref[k]
    hi = ihi_ref[k]
    rows = lax.broadcasted_iota(jnp.int32, (RG, 1), 0)
    valid = (rows >= lo) & (rows < hi)
    pre = jnp.dot(xs_ref[...], ew1_ref[0],
                  preferred_element_type=jnp.float32) + eb1_ref[0]
    hid = pre * jax.nn.sigmoid(pre)
    part = jnp.dot(hid, ew2_ref[0], preferred_element_type=jnp.float32)
    part = part + jnp.where(f == 0, eb2_ref[0], 0.0)
    sl = pl.ds(it_ref[k] * RG, RG)
    prev = jnp.where(f == 0, 0.0, acc_ref[sl, :])
    acc_ref[sl, :] = jnp.where(valid, prev + part, acc_ref[sl, :])
    out_ref[...] = acc_ref[sl, :]


def _gg_call(xs, ew1, eb1, ew2, eb2, ie, it, ilo, ihi):
    grid_spec = pltpu.PrefetchScalarGridSpec(
        num_scalar_prefetch=4,
        grid=(NF, NI),
        in_specs=[
            pl.BlockSpec((RG, D), lambda f, k, ie, it, lo, hi: (it[k], 0)),
            pl.BlockSpec((1, D, FB), lambda f, k, ie, it, lo, hi:
                         (ie[k], 0, f)),
            pl.BlockSpec((1, 1, FB), lambda f, k, ie, it, lo, hi:
                         (ie[k], 0, f)),
            pl.BlockSpec((1, FB, D), lambda f, k, ie, it, lo, hi:
                         (ie[k], f, 0)),
            pl.BlockSpec((1, 1, D), lambda f, k, ie, it, lo, hi:
                         (ie[k], 0, 0)),
        ],
        out_specs=pl.BlockSpec((RG, D), lambda f, k, ie, it, lo, hi:
                               (it[k], 0)),
        scratch_shapes=[pltpu.VMEM((NP, D), jnp.float32)],
    )
    return pl.pallas_call(
        _gg_body,
        grid_spec=grid_spec,
        out_shape=jax.ShapeDtypeStruct((NP, D), jnp.float32),
    )(ie, it, ilo, ihi, xs, ew1, eb1.reshape(E, 1, DFF), ew2,
      eb2.reshape(E, 1, D))


def _comb_body(x_ref, o1_ref, o2_ref, w_ref, out_ref):
    ei = lax.broadcasted_iota(jnp.int32, (BS, E), 1)
    wp = w_ref[...]
    w1 = jnp.sum(jnp.where(ei == 0, wp, 0.0), axis=1, keepdims=True)
    w2 = jnp.sum(jnp.where(ei == 1, wp, 0.0), axis=1, keepdims=True)
    out_ref[...] = x_ref[...] + w1 * o1_ref[...] + w2 * o2_ref[...]


def _comb_call(x2d, og, wpad):
    return pl.pallas_call(
        _comb_body,
        grid=(NS,),
        in_specs=[
            pl.BlockSpec((BS, D), lambda i: (i, 0)),
            pl.BlockSpec((BS, D), lambda i: (i, 0)),
            pl.BlockSpec((BS, D), lambda i: (i + NS, 0)),
            pl.BlockSpec((BS, E), lambda i: (i, 0)),
        ],
        out_specs=pl.BlockSpec((BS, D), lambda i: (i, 0)),
        out_shape=jax.ShapeDtypeStruct((S, D), jnp.float32),
    )(x2d, og, og, wpad)


_QPERM = _deint_perm(HQ)
_KPERM = _deint_perm(HKV)


def kernel(x, norm1_w, q_w, k_w, v_w, o_w, norm2_w, gate_w, gate_b,
           nz_w, nz_b, ew1, eb1, ew2, eb2, noise):
    x2d = x.reshape(S, D)
    cq, sq = _rope_tables(D)
    ck, sk = _rope_tables(HKV * HD)
    qwp = q_w[:, _QPERM]
    kwp = k_w[:, _KPERM]

    q, k, v = _qkv_call(x2d, norm1_w, qwp, kwp, v_w, cq, sq, ck, sk)
    qh = q.reshape(S, 2, HQ, 32).transpose(2, 0, 1, 3).reshape(HQ, S, HD)
    kh = k.reshape(S, 2, HKV, 32).transpose(2, 0, 1, 3).reshape(HKV, S, HD)
    vh = v.reshape(S, HKV, HD).transpose(1, 0, 2)
    attn = _attn_call(qh, kh, vh)
    attn2d = attn.transpose(1, 0, 2).reshape(S, D)
    xa = _oproj_call(x2d, attn2d, o_w)

    h2, wpad, aux, pos, ie, it, ilo, ihi = _router_call(
        xa, norm2_w, gate_w, gate_b, nz_w, nz_b, noise)
    pos1 = pos.reshape(NP)
    xs = _sc_scatter_rows(h2, pos1)
    out_sorted = _gg_call(xs, ew1, eb1, ew2, eb2,
                          ie.reshape(NI), it.reshape(NI),
                          ilo.reshape(NI), ihi.reshape(NI))
    og = _sc_gather_rows(out_sorted, pos1)
    out = _comb_call(xa, og, wpad)
    return out.reshape(1, S, D), aux[0, 0]


# speedup vs baseline: 2.4679x; 1.1331x over previous
import functools

import jax
import jax.numpy as jnp
import numpy as np
from jax import lax
from jax.experimental import pallas as pl
from jax.experimental.pallas import tpu as pltpu
from jax.experimental.pallas import tpu_sc as plsc

S, D = 2048, 768
HQ, HKV, HD = 12, 4, 64
E, DFF = 8, 3072
BS = 256
NS = S // BS
NF = 1
FB = DFF // NF
NP = 2 * S
RG = 256
NT = NP // RG
NI = NT + E
NEG = -1e30


def _rope_tables(n_cols):
    half = n_cols // 2
    inv = 1.0 / (10000.0 ** (np.arange(0, HD, 2, dtype=np.float64) / HD))
    t = np.arange(S, dtype=np.float64)[:, None]
    f = np.tile(t * inv[None, :], (1, half // 32))
    f = np.concatenate([f, f], axis=1)
    return (jnp.asarray(np.cos(f), jnp.float32),
            jnp.asarray(np.sin(f), jnp.float32))


def _deint_perm(n_heads):
    cols = []
    for halfsel in (0, 1):
        for h in range(n_heads):
            for i in range(32):
                cols.append(h * HD + 2 * i + halfsel)
    return np.asarray(cols, np.int32)


def _qkv_body(x_ref, n1_ref, qw_ref, kw_ref, vw_ref, cq_ref, sq_ref,
              ck_ref, sk_ref, q_ref, k_ref, v_ref):
    x = x_ref[...]
    h = x * lax.rsqrt(jnp.mean(x * x, axis=-1, keepdims=True) + 1e-8)
    h = h * n1_ref[...]
    q = jnp.dot(h, qw_ref[...], preferred_element_type=jnp.float32)
    k = jnp.dot(h, kw_ref[...], preferred_element_type=jnp.float32)
    v = jnp.dot(h, vw_ref[...], preferred_element_type=jnp.float32)
    qe, qo = q[:, :D // 2], q[:, D // 2:]
    qsw = jnp.concatenate([-qo, qe], axis=1)
    q_ref[...] = q * cq_ref[...] + qsw * sq_ref[...]
    kd = HKV * HD
    ke, ko = k[:, :kd // 2], k[:, kd // 2:]
    ksw = jnp.concatenate([-ko, ke], axis=1)
    k_ref[...] = k * ck_ref[...] + ksw * sk_ref[...]
    v_ref[...] = v


def _qkv_call(x2d, n1, qwp, kwp, vw, cq, sq, ck, sk):
    kd = HKV * HD
    return pl.pallas_call(
        _qkv_body,
        grid=(NS,),
        in_specs=[
            pl.BlockSpec((BS, D), lambda i: (i, 0)),
            pl.BlockSpec((1, D), lambda i: (0, 0)),
            pl.BlockSpec((D, D), lambda i: (0, 0)),
            pl.BlockSpec((D, kd), lambda i: (0, 0)),
            pl.BlockSpec((D, kd), lambda i: (0, 0)),
            pl.BlockSpec((BS, D), lambda i: (i, 0)),
            pl.BlockSpec((BS, D), lambda i: (i, 0)),
            pl.BlockSpec((BS, kd), lambda i: (i, 0)),
            pl.BlockSpec((BS, kd), lambda i: (i, 0)),
        ],
        out_specs=[
            pl.BlockSpec((BS, D), lambda i: (i, 0)),
            pl.BlockSpec((BS, kd), lambda i: (i, 0)),
            pl.BlockSpec((BS, kd), lambda i: (i, 0)),
        ],
        out_shape=[
            jax.ShapeDtypeStruct((S, D), jnp.float32),
            jax.ShapeDtypeStruct((S, kd), jnp.float32),
            jax.ShapeDtypeStruct((S, kd), jnp.float32),
        ],
    )(x2d, n1.reshape(1, D), qwp, kwp, vw, cq, sq, ck, sk)


def _attn_body(q_ref, k_ref, v_ref, o_ref):
    qb = pl.program_id(1)
    q = q_ref[0]
    k = k_ref[0]
    v = v_ref[0]
    s = lax.dot_general(q, k, (((1,), (1,)), ((), ())),
                        preferred_element_type=jnp.float32)
    s = s * (1.0 / np.sqrt(HD))
    qi = qb * BS + lax.broadcasted_iota(jnp.int32, (BS, S), 0)
    ki = lax.broadcasted_iota(jnp.int32, (BS, S), 1)
    s = jnp.where(ki <= qi, s, NEG)
    m = jnp.max(s, axis=1, keepdims=True)
    p = jnp.exp(s - m)
    l = jnp.sum(p, axis=1, keepdims=True)
    o = jnp.dot(p, v, preferred_element_type=jnp.float32)
    o_ref[0] = o / l


def _attn_call(qh, kh, vh):
    rep = HQ // HKV
    return pl.pallas_call(
        _attn_body,
        grid=(HQ, NS),
        in_specs=[
            pl.BlockSpec((1, BS, HD), lambda h, i: (h, i, 0)),
            pl.BlockSpec((1, S, HD), lambda h, i: (h // rep, 0, 0)),
            pl.BlockSpec((1, S, HD), lambda h, i: (h // rep, 0, 0)),
        ],
        out_specs=pl.BlockSpec((1, BS, HD), lambda h, i: (h, i, 0)),
        out_shape=jax.ShapeDtypeStruct((HQ, S, HD), jnp.float32),
    )(qh, kh, vh)


def _oproj_body(x_ref, a_ref, ow_ref, o_ref):
    o_ref[...] = x_ref[...] + jnp.dot(a_ref[...], ow_ref[...],
                                      preferred_element_type=jnp.float32)


def _oproj_call(x2d, attn2d, ow):
    return pl.pallas_call(
        _oproj_body,
        grid=(NS,),
        in_specs=[
            pl.BlockSpec((BS, D), lambda i: (i, 0)),
            pl.BlockSpec((BS, D), lambda i: (i, 0)),
            pl.BlockSpec((D, D), lambda i: (0, 0)),
        ],
        out_specs=pl.BlockSpec((BS, D), lambda i: (i, 0)),
        out_shape=jax.ShapeDtypeStruct((S, D), jnp.float32),
    )(x2d, attn2d, ow)


def _router_body(x_ref, n2_ref, gw_ref, gb_ref, nw_ref, nb_ref, noise_ref,
                 h2_ref, w_ref, aux_ref, pos_ref, ie_ref, it_ref,
                 ilo_ref, ihi_ref):
    x = x_ref[...]
    h2 = x * lax.rsqrt(jnp.mean(x * x, axis=-1, keepdims=True) + 1e-8)
    h2 = h2 * n2_ref[...]
    h2_ref[...] = h2
    hp = lax.Precision.HIGHEST
    h2b = h2.astype(jnp.bfloat16)
    logits = (jnp.dot(h2b, gw_ref[...].astype(jnp.bfloat16),
                      preferred_element_type=jnp.float32)
              + gb_ref[...] + noise_ref[...]
              + jnp.dot(h2b, nw_ref[...].astype(jnp.bfloat16),
                        preferred_element_type=jnp.float32)
              + nb_ref[...])
    ei = lax.broadcasted_iota(jnp.int32, (S, E), 1)
    m1 = jnp.max(logits, axis=1, keepdims=True)
    i1 = jnp.min(jnp.where(logits == m1, ei, E), axis=1, keepdims=True)
    oh1 = (ei == i1)
    lm = jnp.where(oh1, NEG, logits)
    m2 = jnp.max(lm, axis=1, keepdims=True)
    i2 = jnp.min(jnp.where(lm == m2, ei, E), axis=1, keepdims=True)
    oh2 = (ei == i2)
    sel = oh1 | oh2
    z = jnp.where(sel, jnp.exp(logits - m1), 0.0)
    scores = z / jnp.sum(z, axis=1, keepdims=True)
    w1 = jnp.sum(jnp.where(oh1, scores, 0.0), axis=1, keepdims=True)
    w2 = jnp.sum(jnp.where(oh2, scores, 0.0), axis=1, keepdims=True)
    w_ref[...] = jnp.where(ei == 0, w1, 0.0) + jnp.where(ei == 1, w2, 0.0)
    imp = jnp.mean(scores, axis=0, keepdims=True)
    u = 1.0 / E
    aux_ref[...] = jnp.full(
        (1, 1), jnp.sum(u * (jnp.log(u) - jnp.log(imp + 1e-8))), jnp.float32)

    o1f = oh1.astype(jnp.float32)
    o2f = oh2.astype(jnp.float32)
    counts = (jnp.sum(o1f, axis=0, keepdims=True)
              + jnp.sum(o2f, axis=0, keepdims=True))
    e0 = lax.broadcasted_iota(jnp.int32, (E, E), 0)
    e1 = lax.broadcasted_iota(jnp.int32, (E, E), 1)
    u8s = (e0 < e1).astype(jnp.float32)
    u8i = (e0 <= e1).astype(jnp.float32)
    off = jnp.dot(counts, u8s, preferred_element_type=jnp.float32, precision=hp)
    gin = jnp.dot(counts, u8i, preferred_element_type=jnp.float32, precision=hp)
    c0 = lax.broadcasted_iota(jnp.int32, (128, 128), 0)
    c1 = lax.broadcasted_iota(jnp.int32, (128, 128), 1)
    ltri = (c1 < c0).astype(jnp.float32)
    carry = jnp.zeros((1, E), jnp.float32)
    nch = S // 128
    for half, ohf in ((0, o1f), (1, o2f)):
        for c in range(nch):
            ch = ohf[c * 128:(c + 1) * 128, :]
            excl = jnp.dot(ltri, ch, preferred_element_type=jnp.float32, precision=hp)
            excl = excl + carry
            posc = jnp.sum((excl + off) * ch, axis=1, keepdims=True)
            pos_ref[half * S + c * 128:half * S + (c + 1) * 128, :] = (
                posc.astype(jnp.int32))
            carry = carry + jnp.sum(ch, axis=0, keepdims=True)

    cnt = counts.astype(jnp.int32)
    offi = off.astype(jnp.int32)
    gini = gin.astype(jnp.int32)
    a = offi // RG
    b = (gini - 1) // RG
    n = jnp.where(cnt > 0, b - a + 1, 0)
    nf = n.astype(jnp.float32)
    sexf = jnp.dot(nf, u8s, preferred_element_type=jnp.float32, precision=hp)
    sex = sexf.astype(jnp.int32)
    sin = sex + n
    total = jnp.sum(n)
    erow = lax.broadcasted_iota(jnp.int32, (1, E), 1)
    last_e = jnp.max(jnp.where(n > 0, erow, -1))
    kk = lax.broadcasted_iota(jnp.int32, (NI, 1), 0)
    eraw = jnp.sum((kk >= sin).astype(jnp.int32), axis=1, keepdims=True)
    ek = jnp.minimum(eraw, last_e)
    ohk = (ek == lax.broadcasted_iota(jnp.int32, (NI, E), 1)).astype(
        jnp.int32)
    aK = jnp.sum(ohk * a, axis=1, keepdims=True)
    sK = jnp.sum(ohk * sex, axis=1, keepdims=True)
    oK = jnp.sum(ohk * offi, axis=1, keepdims=True)
    gK = jnp.sum(ohk * gini, axis=1, keepdims=True)
    b_last = jnp.sum(jnp.where(erow == last_e, b, 0))
    valid = kk < total
    tK = jnp.where(valid, aK + kk - sK, b_last)
    lo = jnp.clip(oK - tK * RG, 0, RG)
    hi = jnp.clip(gK - tK * RG, 0, RG)
    ie_ref[...] = ek
    it_ref[...] = tK
    ilo_ref[...] = jnp.where(valid, lo, 0)
    ihi_ref[...] = jnp.where(valid, hi, 0)


def _router_call(x2d, n2, gw, gb, nw, nb, noise):
    return pl.pallas_call(
        _router_body,
        out_shape=[
            jax.ShapeDtypeStruct((S, D), jnp.float32),
            jax.ShapeDtypeStruct((S, E), jnp.float32),
            jax.ShapeDtypeStruct((1, 1), jnp.float32),
            jax.ShapeDtypeStruct((NP, 1), jnp.int32),
            jax.ShapeDtypeStruct((NI, 1), jnp.int32),
            jax.ShapeDtypeStruct((NI, 1), jnp.int32),
            jax.ShapeDtypeStruct((NI, 1), jnp.int32),
            jax.ShapeDtypeStruct((NI, 1), jnp.int32),
        ],
    )(x2d, n2.reshape(1, D), gw, gb.reshape(1, E), nw, nb.reshape(1, E),
      noise.reshape(S, E))


def _sc_mesh():
    return plsc.VectorSubcoreMesh(core_axis_name="c", subcore_axis_name="s",
                                  num_cores=2, num_subcores=16)


def _sc_scatter_rows(src, pos):
    ch = NP // 32

    @functools.partial(
        pl.kernel,
        out_type=jax.ShapeDtypeStruct((NP, D), jnp.float32),
        mesh=_sc_mesh(),
        scratch_types=[pltpu.VMEM((ch,), jnp.int32),
                       pltpu.VMEM((ch, D), jnp.float32),
                       pltpu.SemaphoreType.DMA],
    )
    def k(src_hbm, pos_hbm, out_hbm, idx_v, rows_v, sem):
        wid = lax.axis_index("s") * 2 + lax.axis_index("c")
        base = wid * ch
        pltpu.sync_copy(pos_hbm.at[pl.ds(base, ch)], idx_v)
        pltpu.sync_copy(src_hbm.at[pl.ds((wid % (S // ch)) * ch, ch)],
                        rows_v)
        pltpu.async_copy(rows_v, out_hbm.at[idx_v], sem).wait()

    return k(src, pos)


def _sc_gather_rows(table, idx):
    nrows = idx.shape[0]
    ch = nrows // 32

    @functools.partial(
        pl.kernel,
        out_type=jax.ShapeDtypeStruct((nrows, D), jnp.float32),
        mesh=_sc_mesh(),
        scratch_types=[pltpu.VMEM((ch,), jnp.int32),
                       pltpu.VMEM((ch, D), jnp.float32),
                       pltpu.SemaphoreType.DMA],
    )
    def k(tab_hbm, idx_hbm, out_hbm, idx_v, rows_v, sem):
        wid = lax.axis_index("s") * 2 + lax.axis_index("c")
        base = wid * ch
        pltpu.sync_copy(idx_hbm.at[pl.ds(base, ch)], idx_v)
        pltpu.async_copy(tab_hbm.at[idx_v], rows_v, sem).wait()
        pltpu.sync_copy(rows_v, out_hbm.at[pl.ds(base, ch)])

    return k(table, idx)


def _gg_body(ie_ref, it_ref, ilo_ref, ihi_ref, xs_ref, ew1_ref, eb1_ref,
             ew2_ref, eb2_ref, out_ref, acc_ref):
    f = pl.program_id(0)
    k = pl.program_id(1)
    lo = ilo_ref[k]
    hi = ihi_ref[k]
    rows = lax.broadcasted_iota(jnp.int32, (RG, 1), 0)
    valid = (rows >= lo) & (rows < hi)
    pre = jnp.dot(xs_ref[...], ew1_ref[0],
                  preferred_element_type=jnp.float32) + eb1_ref[0]
    hid = pre * jax.nn.sigmoid(pre)
    part = jnp.dot(hid, ew2_ref[0], preferred_element_type=jnp.float32)
    part = part + jnp.where(f == 0, eb2_ref[0], 0.0)
    sl = pl.ds(it_ref[k] * RG, RG)
    prev = jnp.where(f == 0, 0.0, acc_ref[sl, :])
    acc_ref[sl, :] = jnp.where(valid, prev + part, acc_ref[sl, :])
    out_ref[...] = acc_ref[sl, :]


def _gg_call(xs, ew1, eb1, ew2, eb2, ie, it, ilo, ihi):
    grid_spec = pltpu.PrefetchScalarGridSpec(
        num_scalar_prefetch=4,
        grid=(NF, NI),
        in_specs=[
            pl.BlockSpec((RG, D), lambda f, k, ie, it, lo, hi: (it[k], 0)),
            pl.BlockSpec((1, D, FB), lambda f, k, ie, it, lo, hi:
                         (ie[k], 0, f)),
            pl.BlockSpec((1, 1, FB), lambda f, k, ie, it, lo, hi:
                         (ie[k], 0, f)),
            pl.BlockSpec((1, FB, D), lambda f, k, ie, it, lo, hi:
                         (ie[k], f, 0)),
            pl.BlockSpec((1, 1, D), lambda f, k, ie, it, lo, hi:
                         (ie[k], 0, 0)),
        ],
        out_specs=pl.BlockSpec((RG, D), lambda f, k, ie, it, lo, hi:
                               (it[k], 0)),
        scratch_shapes=[pltpu.VMEM((NP, D), jnp.float32)],
    )
    return pl.pallas_call(
        _gg_body,
        grid_spec=grid_spec,
        out_shape=jax.ShapeDtypeStruct((NP, D), jnp.float32),
    )(ie, it, ilo, ihi, xs, ew1, eb1.reshape(E, 1, DFF), ew2,
      eb2.reshape(E, 1, D))


def _comb_body(x_ref, o1_ref, o2_ref, w_ref, out_ref):
    ei = lax.broadcasted_iota(jnp.int32, (BS, E), 1)
    wp = w_ref[...]
    w1 = jnp.sum(jnp.where(ei == 0, wp, 0.0), axis=1, keepdims=True)
    w2 = jnp.sum(jnp.where(ei == 1, wp, 0.0), axis=1, keepdims=True)
    out_ref[...] = x_ref[...] + w1 * o1_ref[...] + w2 * o2_ref[...]


def _comb_call(x2d, og, wpad):
    return pl.pallas_call(
        _comb_body,
        grid=(NS,),
        in_specs=[
            pl.BlockSpec((BS, D), lambda i: (i, 0)),
            pl.BlockSpec((BS, D), lambda i: (i, 0)),
            pl.BlockSpec((BS, D), lambda i: (i + NS, 0)),
            pl.BlockSpec((BS, E), lambda i: (i, 0)),
        ],
        out_specs=pl.BlockSpec((BS, D), lambda i: (i, 0)),
        out_shape=jax.ShapeDtypeStruct((S, D), jnp.float32),
    )(x2d, og, og, wpad)


_QPERM = _deint_perm(HQ)
_KPERM = _deint_perm(HKV)


def kernel(x, norm1_w, q_w, k_w, v_w, o_w, norm2_w, gate_w, gate_b,
           nz_w, nz_b, ew1, eb1, ew2, eb2, noise):
    x2d = x.reshape(S, D)
    cq, sq = _rope_tables(D)
    ck, sk = _rope_tables(HKV * HD)
    qwp = q_w[:, _QPERM]
    kwp = k_w[:, _KPERM]

    q, k, v = _qkv_call(x2d, norm1_w, qwp, kwp, v_w, cq, sq, ck, sk)
    qh = q.reshape(S, 2, HQ, 32).transpose(2, 0, 1, 3).reshape(HQ, S, HD)
    kh = k.reshape(S, 2, HKV, 32).transpose(2, 0, 1, 3).reshape(HKV, S, HD)
    vh = v.reshape(S, HKV, HD).transpose(1, 0, 2)
    attn = _attn_call(qh, kh, vh)
    attn2d = attn.transpose(1, 0, 2).reshape(S, D)
    xa = _oproj_call(x2d, attn2d, o_w)

    h2, wpad, aux, pos, ie, it, ilo, ihi = _router_call(
        xa, norm2_w, gate_w, gate_b, nz_w, nz_b, noise)
    pos1 = pos.reshape(NP)
    xs = _sc_scatter_rows(h2, pos1)
    out_sorted = _gg_call(xs, ew1, eb1, ew2, eb2,
                          ie.reshape(NI), it.reshape(NI),
                          ilo.reshape(NI), ihi.reshape(NI))
    og = _sc_gather_rows(out_sorted, pos1)
    out = _comb_call(xa, og, wpad)
    return out.reshape(1, S, D), aux[0, 0]


# speedup vs baseline: 2.5733x; 1.0427x over previous
import functools

import jax
import jax.numpy as jnp
import numpy as np
from jax import lax
from jax.experimental import pallas as pl
from jax.experimental.pallas import tpu as pltpu
from jax.experimental.pallas import tpu_sc as plsc

S, D = 2048, 768
HQ, HKV, HD = 12, 4, 64
E, DFF = 8, 3072
BS = 256
NS = S // BS
NF = 1
FB = DFF // NF
NP = 2 * S
RG = 256
NT = NP // RG
NI = NT + E
NEG = -1e30


def _rope_tables(n_cols):
    half = n_cols // 2
    inv = 1.0 / (10000.0 ** (np.arange(0, HD, 2, dtype=np.float64) / HD))
    t = np.arange(S, dtype=np.float64)[:, None]
    f = np.tile(t * inv[None, :], (1, half // 32))
    f = np.concatenate([f, f], axis=1)
    return (jnp.asarray(np.cos(f), jnp.float32),
            jnp.asarray(np.sin(f), jnp.float32))


def _deint_perm(n_heads):
    cols = []
    for halfsel in (0, 1):
        for h in range(n_heads):
            for i in range(32):
                cols.append(h * HD + 2 * i + halfsel)
    return np.asarray(cols, np.int32)


def _qkv_body(x_ref, n1_ref, qw_ref, kw_ref, vw_ref, cq_ref, sq_ref,
              ck_ref, sk_ref, q_ref, k_ref, v_ref):
    x = x_ref[...]
    h = x * lax.rsqrt(jnp.mean(x * x, axis=-1, keepdims=True) + 1e-8)
    h = h * n1_ref[...]
    q = jnp.dot(h, qw_ref[...], preferred_element_type=jnp.float32)
    k = jnp.dot(h, kw_ref[...], preferred_element_type=jnp.float32)
    v = jnp.dot(h, vw_ref[...], preferred_element_type=jnp.float32)
    qe, qo = q[:, :D // 2], q[:, D // 2:]
    qsw = jnp.concatenate([-qo, qe], axis=1)
    q_ref[...] = q * cq_ref[...] + qsw * sq_ref[...]
    kd = HKV * HD
    ke, ko = k[:, :kd // 2], k[:, kd // 2:]
    ksw = jnp.concatenate([-ko, ke], axis=1)
    k_ref[...] = k * ck_ref[...] + ksw * sk_ref[...]
    v_ref[...] = v


def _qkv_call(x2d, n1, qwp, kwp, vw, cq, sq, ck, sk):
    kd = HKV * HD
    return pl.pallas_call(
        _qkv_body,
        grid=(NS,),
        in_specs=[
            pl.BlockSpec((BS, D), lambda i: (i, 0)),
            pl.BlockSpec((1, D), lambda i: (0, 0)),
            pl.BlockSpec((D, D), lambda i: (0, 0)),
            pl.BlockSpec((D, kd), lambda i: (0, 0)),
            pl.BlockSpec((D, kd), lambda i: (0, 0)),
            pl.BlockSpec((BS, D), lambda i: (i, 0)),
            pl.BlockSpec((BS, D), lambda i: (i, 0)),
            pl.BlockSpec((BS, kd), lambda i: (i, 0)),
            pl.BlockSpec((BS, kd), lambda i: (i, 0)),
        ],
        out_specs=[
            pl.BlockSpec((BS, D), lambda i: (i, 0)),
            pl.BlockSpec((BS, kd), lambda i: (i, 0)),
            pl.BlockSpec((BS, kd), lambda i: (i, 0)),
        ],
        out_shape=[
            jax.ShapeDtypeStruct((S, D), jnp.float32),
            jax.ShapeDtypeStruct((S, kd), jnp.float32),
            jax.ShapeDtypeStruct((S, kd), jnp.float32),
        ],
    )(x2d, n1.reshape(1, D), qwp, kwp, vw, cq, sq, ck, sk)


CH = 512


def _attn_body(q_ref, k_ref, v_ref, o_ref):
    qb = pl.program_id(1)
    q = q_ref[0]
    qi = qb * BS + lax.broadcasted_iota(jnp.int32, (BS, CH), 0)
    ki0 = lax.broadcasted_iota(jnp.int32, (BS, CH), 1)
    scale = 1.0 / np.sqrt(HD)

    def chunk(c, carry):
        m, l, acc = carry
        kc = k_ref[0, pl.ds(c * CH, CH), :]
        vc = v_ref[0, pl.ds(c * CH, CH), :]
        s = lax.dot_general(q, kc, (((1,), (1,)), ((), ())),
                            preferred_element_type=jnp.float32) * scale
        s = jnp.where(c * CH + ki0 <= qi, s, NEG)
        mn = jnp.maximum(m, jnp.max(s, axis=1, keepdims=True))
        p = jnp.exp(s - mn)
        corr = jnp.exp(m - mn)
        l = l * corr + jnp.sum(p, axis=1, keepdims=True)
        acc = acc * corr + jnp.dot(p, vc, preferred_element_type=jnp.float32)
        return mn, l, acc

    m0 = jnp.full((BS, 1), NEG, jnp.float32)
    l0 = jnp.zeros((BS, 1), jnp.float32)
    a0 = jnp.zeros((BS, HD), jnp.float32)
    nch = (qb * BS) // CH + 1
    m, l, acc = lax.fori_loop(0, nch, chunk, (m0, l0, a0))
    o_ref[0] = acc / l


def _attn_call(qh, kh, vh):
    rep = HQ // HKV
    return pl.pallas_call(
        _attn_body,
        grid=(HQ, NS),
        in_specs=[
            pl.BlockSpec((1, BS, HD), lambda h, i: (h, i, 0)),
            pl.BlockSpec((1, S, HD), lambda h, i: (h // rep, 0, 0)),
            pl.BlockSpec((1, S, HD), lambda h, i: (h // rep, 0, 0)),
        ],
        out_specs=pl.BlockSpec((1, BS, HD), lambda h, i: (h, i, 0)),
        out_shape=jax.ShapeDtypeStruct((HQ, S, HD), jnp.float32),
    )(qh, kh, vh)


def _oproj_body(x_ref, a_ref, ow_ref, o_ref):
    o_ref[...] = x_ref[...] + jnp.dot(a_ref[...], ow_ref[...],
                                      preferred_element_type=jnp.float32)


def _oproj_call(x2d, attn2d, ow):
    return pl.pallas_call(
        _oproj_body,
        grid=(NS,),
        in_specs=[
            pl.BlockSpec((BS, D), lambda i: (i, 0)),
            pl.BlockSpec((BS, D), lambda i: (i, 0)),
            pl.BlockSpec((D, D), lambda i: (0, 0)),
        ],
        out_specs=pl.BlockSpec((BS, D), lambda i: (i, 0)),
        out_shape=jax.ShapeDtypeStruct((S, D), jnp.float32),
    )(x2d, attn2d, ow)


def _router_body(x_ref, n2_ref, gw_ref, gb_ref, nw_ref, nb_ref, noise_ref,
                 h2_ref, w_ref, aux_ref, pos_ref, ie_ref, it_ref,
                 ilo_ref, ihi_ref):
    x = x_ref[...]
    h2 = x * lax.rsqrt(jnp.mean(x * x, axis=-1, keepdims=True) + 1e-8)
    h2 = h2 * n2_ref[...]
    h2_ref[...] = h2
    hp = lax.Precision.HIGHEST
    h2b = h2.astype(jnp.bfloat16)
    logits = (jnp.dot(h2b, gw_ref[...].astype(jnp.bfloat16),
                      preferred_element_type=jnp.float32)
              + gb_ref[...] + noise_ref[...]
              + jnp.dot(h2b, nw_ref[...].astype(jnp.bfloat16),
                        preferred_element_type=jnp.float32)
              + nb_ref[...])
    ei = lax.broadcasted_iota(jnp.int32, (S, E), 1)
    m1 = jnp.max(logits, axis=1, keepdims=True)
    i1 = jnp.min(jnp.where(logits == m1, ei, E), axis=1, keepdims=True)
    oh1 = (ei == i1)
    lm = jnp.where(oh1, NEG, logits)
    m2 = jnp.max(lm, axis=1, keepdims=True)
    i2 = jnp.min(jnp.where(lm == m2, ei, E), axis=1, keepdims=True)
    oh2 = (ei == i2)
    sel = oh1 | oh2
    z = jnp.where(sel, jnp.exp(logits - m1), 0.0)
    scores = z / jnp.sum(z, axis=1, keepdims=True)
    w1 = jnp.sum(jnp.where(oh1, scores, 0.0), axis=1, keepdims=True)
    w2 = jnp.sum(jnp.where(oh2, scores, 0.0), axis=1, keepdims=True)
    w_ref[...] = jnp.where(ei == 0, w1, 0.0) + jnp.where(ei == 1, w2, 0.0)
    imp = jnp.mean(scores, axis=0, keepdims=True)
    u = 1.0 / E
    aux_ref[...] = jnp.full(
        (1, 1), jnp.sum(u * (jnp.log(u) - jnp.log(imp + 1e-8))), jnp.float32)

    o1f = oh1.astype(jnp.float32)
    o2f = oh2.astype(jnp.float32)
    counts = (jnp.sum(o1f, axis=0, keepdims=True)
              + jnp.sum(o2f, axis=0, keepdims=True))
    e0 = lax.broadcasted_iota(jnp.int32, (E, E), 0)
    e1 = lax.broadcasted_iota(jnp.int32, (E, E), 1)
    u8s = (e0 < e1).astype(jnp.float32)
    u8i = (e0 <= e1).astype(jnp.float32)
    off = jnp.dot(counts, u8s, preferred_element_type=jnp.float32, precision=hp)
    gin = jnp.dot(counts, u8i, preferred_element_type=jnp.float32, precision=hp)
    c0 = lax.broadcasted_iota(jnp.int32, (128, 128), 0)
    c1 = lax.broadcasted_iota(jnp.int32, (128, 128), 1)
    ltri = (c1 < c0).astype(jnp.float32)
    carry = jnp.zeros((1, E), jnp.float32)
    nch = S // 128
    for half, ohf in ((0, o1f), (1, o2f)):
        for c in range(nch):
            ch = ohf[c * 128:(c + 1) * 128, :]
            excl = jnp.dot(ltri, ch, preferred_element_type=jnp.float32, precision=hp)
            excl = excl + carry
            posc = jnp.sum((excl + off) * ch, axis=1, keepdims=True)
            pos_ref[half * S + c * 128:half * S + (c + 1) * 128, :] = (
                posc.astype(jnp.int32))
            carry = carry + jnp.sum(ch, axis=0, keepdims=True)

    cnt = counts.astype(jnp.int32)
    offi = off.astype(jnp.int32)
    gini = gin.astype(jnp.int32)
    a = offi // RG
    b = (gini - 1) // RG
    n = jnp.where(cnt > 0, b - a + 1, 0)
    nf = n.astype(jnp.float32)
    sexf = jnp.dot(nf, u8s, preferred_element_type=jnp.float32, precision=hp)
    sex = sexf.astype(jnp.int32)
    sin = sex + n
    total = jnp.sum(n)
    erow = lax.broadcasted_iota(jnp.int32, (1, E), 1)
    last_e = jnp.max(jnp.where(n > 0, erow, -1))
    kk = lax.broadcasted_iota(jnp.int32, (NI, 1), 0)
    eraw = jnp.sum((kk >= sin).astype(jnp.int32), axis=1, keepdims=True)
    ek = jnp.minimum(eraw, last_e)
    ohk = (ek == lax.broadcasted_iota(jnp.int32, (NI, E), 1)).astype(
        jnp.int32)
    aK = jnp.sum(ohk * a, axis=1, keepdims=True)
    sK = jnp.sum(ohk * sex, axis=1, keepdims=True)
    oK = jnp.sum(ohk * offi, axis=1, keepdims=True)
    gK = jnp.sum(ohk * gini, axis=1, keepdims=True)
    b_last = jnp.sum(jnp.where(erow == last_e, b, 0))
    valid = kk < total
    tK = jnp.where(valid, aK + kk - sK, b_last)
    lo = jnp.clip(oK - tK * RG, 0, RG)
    hi = jnp.clip(gK - tK * RG, 0, RG)
    ie_ref[...] = ek
    it_ref[...] = tK
    ilo_ref[...] = jnp.where(valid, lo, 0)
    ihi_ref[...] = jnp.where(valid, hi, 0)


def _router_call(x2d, n2, gw, gb, nw, nb, noise):
    return pl.pallas_call(
        _router_body,
        out_shape=[
            jax.ShapeDtypeStruct((S, D), jnp.float32),
            jax.ShapeDtypeStruct((S, E), jnp.float32),
            jax.ShapeDtypeStruct((1, 1), jnp.float32),
            jax.ShapeDtypeStruct((NP, 1), jnp.int32),
            jax.ShapeDtypeStruct((NI, 1), jnp.int32),
            jax.ShapeDtypeStruct((NI, 1), jnp.int32),
            jax.ShapeDtypeStruct((NI, 1), jnp.int32),
            jax.ShapeDtypeStruct((NI, 1), jnp.int32),
        ],
    )(x2d, n2.reshape(1, D), gw, gb.reshape(1, E), nw, nb.reshape(1, E),
      noise.reshape(S, E))


def _sc_mesh():
    return plsc.VectorSubcoreMesh(core_axis_name="c", subcore_axis_name="s",
                                  num_cores=2, num_subcores=16)


def _sc_scatter_rows(src, pos):
    ch = NP // 32

    @functools.partial(
        pl.kernel,
        out_type=jax.ShapeDtypeStruct((NP, D), jnp.float32),
        mesh=_sc_mesh(),
        scratch_types=[pltpu.VMEM((ch,), jnp.int32),
                       pltpu.VMEM((ch, D), jnp.float32),
                       pltpu.SemaphoreType.DMA],
    )
    def k(src_hbm, pos_hbm, out_hbm, idx_v, rows_v, sem):
        wid = lax.axis_index("s") * 2 + lax.axis_index("c")
        base = wid * ch
        pltpu.sync_copy(pos_hbm.at[pl.ds(base, ch)], idx_v)
        pltpu.sync_copy(src_hbm.at[pl.ds((wid % (S // ch)) * ch, ch)],
                        rows_v)
        pltpu.async_copy(rows_v, out_hbm.at[idx_v], sem).wait()

    return k(src, pos)


def _sc_gather_rows(table, idx):
    nrows = idx.shape[0]
    ch = nrows // 32

    @functools.partial(
        pl.kernel,
        out_type=jax.ShapeDtypeStruct((nrows, D), jnp.float32),
        mesh=_sc_mesh(),
        scratch_types=[pltpu.VMEM((ch,), jnp.int32),
                       pltpu.VMEM((ch, D), jnp.float32),
                       pltpu.SemaphoreType.DMA],
    )
    def k(tab_hbm, idx_hbm, out_hbm, idx_v, rows_v, sem):
        wid = lax.axis_index("s") * 2 + lax.axis_index("c")
        base = wid * ch
        pltpu.sync_copy(idx_hbm.at[pl.ds(base, ch)], idx_v)
        pltpu.async_copy(tab_hbm.at[idx_v], rows_v, sem).wait()
        pltpu.sync_copy(rows_v, out_hbm.at[pl.ds(base, ch)])

    return k(table, idx)


def _gg_body(ie_ref, it_ref, ilo_ref, ihi_ref, xs_ref, ew1_ref, eb1_ref,
             ew2_ref, eb2_ref, out_ref, acc_ref):
    f = pl.program_id(0)
    k = pl.program_id(1)
    lo = ilo_ref[k]
    hi = ihi_ref[k]
    rows = lax.broadcasted_iota(jnp.int32, (RG, 1), 0)
    valid = (rows >= lo) & (rows < hi)
    pre = jnp.dot(xs_ref[...], ew1_ref[0],
                  preferred_element_type=jnp.float32) + eb1_ref[0]
    hid = pre * jax.nn.sigmoid(pre)
    part = jnp.dot(hid, ew2_ref[0], preferred_element_type=jnp.float32)
    part = part + jnp.where(f == 0, eb2_ref[0], 0.0)
    sl = pl.ds(it_ref[k] * RG, RG)
    prev = jnp.where(f == 0, 0.0, acc_ref[sl, :])
    acc_ref[sl, :] = jnp.where(valid, prev + part, acc_ref[sl, :])
    out_ref[...] = acc_ref[sl, :]


def _gg_call(xs, ew1, eb1, ew2, eb2, ie, it, ilo, ihi):
    grid_spec = pltpu.PrefetchScalarGridSpec(
        num_scalar_prefetch=4,
        grid=(NF, NI),
        in_specs=[
            pl.BlockSpec((RG, D), lambda f, k, ie, it, lo, hi: (it[k], 0)),
            pl.BlockSpec((1, D, FB), lambda f, k, ie, it, lo, hi:
                         (ie[k], 0, f)),
            pl.BlockSpec((1, 1, FB), lambda f, k, ie, it, lo, hi:
                         (ie[k], 0, f)),
            pl.BlockSpec((1, FB, D), lambda f, k, ie, it, lo, hi:
                         (ie[k], f, 0)),
            pl.BlockSpec((1, 1, D), lambda f, k, ie, it, lo, hi:
                         (ie[k], 0, 0)),
        ],
        out_specs=pl.BlockSpec((RG, D), lambda f, k, ie, it, lo, hi:
                               (it[k], 0)),
        scratch_shapes=[pltpu.VMEM((NP, D), jnp.float32)],
    )
    return pl.pallas_call(
        _gg_body,
        grid_spec=grid_spec,
        out_shape=jax.ShapeDtypeStruct((NP, D), jnp.float32),
    )(ie, it, ilo, ihi, xs, ew1, eb1.reshape(E, 1, DFF), ew2,
      eb2.reshape(E, 1, D))


def _comb_body(x_ref, o1_ref, o2_ref, w_ref, out_ref):
    ei = lax.broadcasted_iota(jnp.int32, (BS, E), 1)
    wp = w_ref[...]
    w1 = jnp.sum(jnp.where(ei == 0, wp, 0.0), axis=1, keepdims=True)
    w2 = jnp.sum(jnp.where(ei == 1, wp, 0.0), axis=1, keepdims=True)
    out_ref[...] = x_ref[...] + w1 * o1_ref[...] + w2 * o2_ref[...]


def _comb_call(x2d, og, wpad):
    return pl.pallas_call(
        _comb_body,
        grid=(NS,),
        in_specs=[
            pl.BlockSpec((BS, D), lambda i: (i, 0)),
            pl.BlockSpec((BS, D), lambda i: (i, 0)),
            pl.BlockSpec((BS, D), lambda i: (i + NS, 0)),
            pl.BlockSpec((BS, E), lambda i: (i, 0)),
        ],
        out_specs=pl.BlockSpec((BS, D), lambda i: (i, 0)),
        out_shape=jax.ShapeDtypeStruct((S, D), jnp.float32),
    )(x2d, og, og, wpad)


_QPERM = _deint_perm(HQ)
_KPERM = _deint_perm(HKV)


def kernel(x, norm1_w, q_w, k_w, v_w, o_w, norm2_w, gate_w, gate_b,
           nz_w, nz_b, ew1, eb1, ew2, eb2, noise):
    x2d = x.reshape(S, D)
    cq, sq = _rope_tables(D)
    ck, sk = _rope_tables(HKV * HD)
    qwp = q_w[:, _QPERM]
    kwp = k_w[:, _KPERM]

    q, k, v = _qkv_call(x2d, norm1_w, qwp, kwp, v_w, cq, sq, ck, sk)
    qh = q.reshape(S, 2, HQ, 32).transpose(2, 0, 1, 3).reshape(HQ, S, HD)
    kh = k.reshape(S, 2, HKV, 32).transpose(2, 0, 1, 3).reshape(HKV, S, HD)
    vh = v.reshape(S, HKV, HD).transpose(1, 0, 2)
    attn = _attn_call(qh, kh, vh)
    attn2d = attn.transpose(1, 0, 2).reshape(S, D)
    xa = _oproj_call(x2d, attn2d, o_w)

    h2, wpad, aux, pos, ie, it, ilo, ihi = _router_call(
        xa, norm2_w, gate_w, gate_b, nz_w, nz_b, noise)
    pos1 = pos.reshape(NP)
    xs = _sc_scatter_rows(h2, pos1)
    out_sorted = _gg_call(xs, ew1, eb1, ew2, eb2,
                          ie.reshape(NI), it.reshape(NI),
                          ilo.reshape(NI), ihi.reshape(NI))
    og = _sc_gather_rows(out_sorted, pos1)
    out = _comb_call(xa, og, wpad)
    return out.reshape(1, S, D), aux[0, 0]
